# baseline (device time: 12891 ns/iter reference)
import jax
import jax.numpy as jnp
from jax import lax
from jax.experimental import pallas as pl
from jax.experimental.pallas import tpu as pltpu

N_CHUNKS = 4


def kernel(x, pi):
    shard_shape = x.shape
    rows = shard_shape[1]
    cols = shard_shape[2]
    rows_per = rows // N_CHUNKS

    def body(pi_ref, x_ref, out_ref, q_tx, q_rx, s_tx, s_rx,
             qsend, qrecv, ssend, srecv):
        my_x = lax.axis_index("x")
        my_y = lax.axis_index("y")
        my_z = lax.axis_index("z")
        peer_y = pi_ref[my_y]
        peer = (my_x, peer_y, my_z)

        barrier_sem = pltpu.get_barrier_semaphore()
        pl.semaphore_signal(
            barrier_sem, inc=1, device_id=peer,
            device_id_type=pl.DeviceIdType.MESH,
        )
        pl.semaphore_wait(barrier_sem, 1)

        data_rdmas = []
        scale_rdmas = []
        for c in range(N_CHUNKS):
            sl = pl.ds(c * rows_per, rows_per)
            xc = x_ref[0, sl, :]
            m = jnp.max(jnp.abs(xc), axis=1, keepdims=True)
            scale = m * (1.0 / 127.0)
            q_tx[0, sl, :] = jnp.rint(xc * (127.0 / m)).astype(jnp.int8)
            s_tx[c] = scale

            srd = pltpu.make_async_remote_copy(
                src_ref=s_tx.at[c], dst_ref=s_rx.at[c],
                send_sem=ssend.at[c], recv_sem=srecv.at[c],
                device_id=peer, device_id_type=pl.DeviceIdType.MESH,
            )
            srd.start()
            drd = pltpu.make_async_remote_copy(
                src_ref=q_tx.at[:, sl, :], dst_ref=q_rx.at[:, sl, :],
                send_sem=qsend.at[c], recv_sem=qrecv.at[c],
                device_id=peer, device_id_type=pl.DeviceIdType.MESH,
            )
            drd.start()
            scale_rdmas.append(srd)
            data_rdmas.append(drd)

        for c in range(N_CHUNKS):
            sl = pl.ds(c * rows_per, rows_per)
            scale_rdmas[c].wait_recv()
            data_rdmas[c].wait_recv()
            qc = q_rx[0, sl, :].astype(jnp.float32)
            sc = s_rx[c]
            out_ref[0, sl, :] = (qc * sc).astype(jnp.bfloat16)

        for c in range(N_CHUNKS):
            scale_rdmas[c].wait_send()
            data_rdmas[c].wait_send()

    out_shape = jax.ShapeDtypeStruct(shard_shape, jnp.bfloat16)
    return pl.pallas_call(
        body,
        out_shape=out_shape,
        in_specs=[
            pl.BlockSpec(memory_space=pltpu.SMEM),
            pl.BlockSpec(memory_space=pltpu.VMEM),
        ],
        out_specs=pl.BlockSpec(memory_space=pltpu.VMEM),
        scratch_shapes=[
            pltpu.VMEM((1, rows, cols), jnp.int8),
            pltpu.VMEM((1, rows, cols), jnp.int8),
            pltpu.VMEM((N_CHUNKS, rows_per, 1), jnp.float32),
            pltpu.VMEM((N_CHUNKS, rows_per, 1), jnp.float32),
            pltpu.SemaphoreType.DMA((N_CHUNKS,)),
            pltpu.SemaphoreType.DMA((N_CHUNKS,)),
            pltpu.SemaphoreType.DMA((N_CHUNKS,)),
            pltpu.SemaphoreType.DMA((N_CHUNKS,)),
        ],
        compiler_params=pltpu.CompilerParams(collective_id=0),
    )(pi, x)


# device time: 12635 ns/iter; 1.0203x vs baseline; 1.0203x over previous
import jax
import jax.numpy as jnp
from jax import lax
from jax.experimental import pallas as pl
from jax.experimental.pallas import tpu as pltpu

N_CHUNKS = 4


def kernel(x, pi):
    shard_shape = x.shape
    rows = shard_shape[1]
    rows_per = rows // N_CHUNKS

    def body(pi_ref, x_ref, out_ref, comm_ref, send_sems, recv_sems):
        my_x = lax.axis_index("x")
        my_y = lax.axis_index("y")
        my_z = lax.axis_index("z")
        peer_y = pi_ref[my_y]
        peer = (my_x, peer_y, my_z)

        barrier_sem = pltpu.get_barrier_semaphore()
        pl.semaphore_signal(
            barrier_sem, inc=1, device_id=peer,
            device_id_type=pl.DeviceIdType.MESH,
        )
        comm_ref[0, pl.ds(0, rows_per), :] = (
            x_ref[0, pl.ds(0, rows_per), :].astype(jnp.bfloat16)
        )
        pl.semaphore_wait(barrier_sem, 1)

        rdmas = []
        for c in range(N_CHUNKS):
            sl = pl.ds(c * rows_per, rows_per)
            rdma = pltpu.make_async_remote_copy(
                src_ref=comm_ref.at[:, sl, :],
                dst_ref=out_ref.at[:, sl, :],
                send_sem=send_sems.at[c],
                recv_sem=recv_sems.at[c],
                device_id=peer,
                device_id_type=pl.DeviceIdType.MESH,
            )
            rdma.start()
            rdmas.append(rdma)
            if c + 1 < N_CHUNKS:
                nxt = pl.ds((c + 1) * rows_per, rows_per)
                comm_ref[0, nxt, :] = x_ref[0, nxt, :].astype(jnp.bfloat16)
        for rdma in rdmas:
            rdma.wait()

    out_shape = jax.ShapeDtypeStruct(shard_shape, jnp.bfloat16)
    return pl.pallas_call(
        body,
        out_shape=out_shape,
        in_specs=[
            pl.BlockSpec(memory_space=pltpu.SMEM),
            pl.BlockSpec(memory_space=pltpu.VMEM),
        ],
        out_specs=pl.BlockSpec(memory_space=pltpu.VMEM),
        scratch_shapes=[
            pltpu.VMEM(shard_shape, jnp.bfloat16),
            pltpu.SemaphoreType.DMA((N_CHUNKS,)),
            pltpu.SemaphoreType.DMA((N_CHUNKS,)),
        ],
        compiler_params=pltpu.CompilerParams(collective_id=0),
    )(pi, x)
